# Initial kernel scaffold; baseline (speedup 1.0000x reference)
import functools

import jax
import jax.numpy as jnp
from jax import lax
from jax.experimental import pallas as pl
from jax.experimental.pallas import tpu as pltpu
from jax.experimental.pallas import tpu_sc as plsc

N = 10000
NP = 10016
D = 128
E = 320000
N_LAYERS = 8

NW = 32
CHUNK = 128
K = 4
NSTEP = 80
EPT = NSTEP * CHUNK
E_PAD = NW * EPT
RPT = NP // 16

_MESH = plsc.VectorSubcoreMesh(core_axis_name="c", subcore_axis_name="s")


@functools.partial(
    pl.kernel,
    out_type=jax.ShapeDtypeStruct((2, NP, 16), jnp.float32),
    mesh=_MESH,
    scratch_types=[
        [pltpu.VMEM((CHUNK,), jnp.int32) for _ in range(K)],
        pltpu.VMEM((CHUNK, 16), jnp.float32),
        pltpu.VMEM_SHARED((NP, 16), jnp.float32),
        pltpu.SemaphoreType.DMA,
        pltpu.SemaphoreType.DMA,
    ],
)
def _deg_kernel(dst_hbm, ones_hbm, zeros_hbm, out_hbm,
                didx, ones_v, acc, sem_i, sem_s):
    c = lax.axis_index("c")
    s = lax.axis_index("s")
    wid = c * 16 + s
    pltpu.sync_copy(zeros_hbm.at[pl.ds(s * RPT, RPT)], acc.at[pl.ds(s * RPT, RPT)])
    pltpu.sync_copy(ones_hbm, ones_v)
    plsc.subcore_barrier()

    ebase = wid * EPT

    def group(gi, carry):
        base = ebase + gi * (K * CHUNK)
        cps = [
            pltpu.async_copy(dst_hbm.at[pl.ds(base + b * CHUNK, CHUNK)], didx[b], sem_i)
            for b in range(K)
        ]
        for cp in cps:
            cp.wait()
        scs = [pltpu.async_copy(ones_v, acc.at[didx[b]], sem_s, add=True) for b in range(K)]
        for cp in scs:
            cp.wait()
        return carry

    lax.fori_loop(0, NSTEP // K, group, 0)
    plsc.subcore_barrier()
    pltpu.sync_copy(acc.at[pl.ds(s * RPT, RPT)], out_hbm.at[c, pl.ds(s * RPT, RPT)])


@functools.partial(
    pl.kernel,
    out_type=jax.ShapeDtypeStruct((2, NP, D), jnp.float32),
    mesh=_MESH,
    scratch_types=[
        [pltpu.VMEM((CHUNK,), jnp.int32) for _ in range(K)],
        [pltpu.VMEM((CHUNK,), jnp.int32) for _ in range(K)],
        [pltpu.VMEM((CHUNK, D), jnp.float32) for _ in range(K)],
        pltpu.VMEM_SHARED((NP, D), jnp.float32),
        pltpu.SemaphoreType.DMA,
        pltpu.SemaphoreType.DMA,
        pltpu.SemaphoreType.DMA,
    ],
)
def _edge_kernel(src_hbm, dst_hbm, table_hbm, zeros_hbm, out_hbm,
                 sidx, didx, rows, acc, sem_i, sem_g, sem_s):
    c = lax.axis_index("c")
    s = lax.axis_index("s")
    wid = c * 16 + s
    pltpu.sync_copy(zeros_hbm.at[pl.ds(s * RPT, RPT)], acc.at[pl.ds(s * RPT, RPT)])
    plsc.subcore_barrier()

    ebase = wid * EPT

    def group(gi, carry):
        base = ebase + gi * (K * CHUNK)
        cps = []
        for b in range(K):
            cps.append(pltpu.async_copy(src_hbm.at[pl.ds(base + b * CHUNK, CHUNK)], sidx[b], sem_i))
            cps.append(pltpu.async_copy(dst_hbm.at[pl.ds(base + b * CHUNK, CHUNK)], didx[b], sem_i))
        for cp in cps:
            cp.wait()
        gcs = [pltpu.async_copy(table_hbm.at[sidx[b]], rows[b], sem_g) for b in range(K)]
        for cp in gcs:
            cp.wait()
        scs = [pltpu.async_copy(rows[b], acc.at[didx[b]], sem_s, add=True) for b in range(K)]
        for cp in scs:
            cp.wait()
        return carry

    lax.fori_loop(0, NSTEP // K, group, 0)
    plsc.subcore_barrier()
    pltpu.sync_copy(acc.at[pl.ds(s * RPT, RPT)], out_hbm.at[c, pl.ds(s * RPT, RPT)])


def _dis(degp_ref):
    deg = degp_ref[0, :, 0:1] + degp_ref[1, :, 0:1] + 1.0
    return lax.rsqrt(deg)


def _first_body(x_ref, w_ref, degp_ref, o_ref):
    dis = _dis(degp_ref)
    g = jnp.dot(x_ref[...], w_ref[...], preferred_element_type=jnp.float32)
    o_ref[...] = g * dis


def _mid_body(s_ref, g_ref, degp_ref, w_ref, b_ref, o_ref):
    dis = _dis(degp_ref)
    h = jnp.maximum(dis * (s_ref[0] + s_ref[1] + g_ref[...]) + b_ref[...], 0.0)
    o_ref[...] = jnp.dot(h, w_ref[...], preferred_element_type=jnp.float32) * dis


def _final_body(s_ref, g_ref, degp_ref, b_ref, wc_ref, bc_ref, o_ref):
    dis = _dis(degp_ref)
    h = jnp.maximum(dis * (s_ref[0] + s_ref[1] + g_ref[...]) + b_ref[...], 0.0)
    o_ref[...] = jnp.dot(h, wc_ref[...], preferred_element_type=jnp.float32) + bc_ref[...]


_first_tc = pl.pallas_call(
    _first_body, out_shape=jax.ShapeDtypeStruct((NP, D), jnp.float32))
_mid_tc = pl.pallas_call(
    _mid_body, out_shape=jax.ShapeDtypeStruct((NP, D), jnp.float32))
_final_tc = pl.pallas_call(
    _final_body, out_shape=jax.ShapeDtypeStruct((NP, D), jnp.float32))


def kernel(x, edge_index, Ws, bs, Wc, bc):
    src = edge_index[0]
    dst = edge_index[1]
    pad = E_PAD - E
    src_p = jnp.concatenate([src, jnp.full((pad,), N, jnp.int32)])
    dst_p = jnp.concatenate([dst, jnp.full((pad,), N, jnp.int32)])

    xp = jnp.pad(x, ((0, NP - N), (0, 0)))
    zeros16 = jnp.zeros((NP, 16), jnp.float32)
    zerosD = jnp.zeros((NP, D), jnp.float32)
    ones16 = jnp.ones((CHUNK, 16), jnp.float32)
    wc_pad = jnp.pad(Wc, ((0, 0), (0, D - Wc.shape[1])))
    bc_pad = jnp.pad(bc, (0, D - bc.shape[0])).reshape(1, D)

    degp = _deg_kernel(dst_p, ones16, zeros16)

    g = _first_tc(xp, Ws[0], degp)
    for i in range(1, N_LAYERS):
        sp = _edge_kernel(src_p, dst_p, g, zerosD)
        g = _mid_tc(sp, g, degp, Ws[i], bs[i - 1].reshape(1, D))
    sp = _edge_kernel(src_p, dst_p, g, zerosD)
    out_full = _final_tc(sp, g, degp, bs[N_LAYERS - 1].reshape(1, D), wc_pad, bc_pad)
    return out_full[:N, :Wc.shape[1]]


# speedup vs baseline: 4.7673x; 4.7673x over previous
import functools

import jax
import jax.numpy as jnp
from jax import lax
from jax.experimental import pallas as pl
from jax.experimental.pallas import tpu as pltpu
from jax.experimental.pallas import tpu_sc as plsc

N = 10000
NP = 10112
D = 128
E = 320000
N_LAYERS = 8

NW = 32
CHUNK = 128
K = 2
NSTEP = 80
EPT = NSTEP * CHUNK
E_PAD = NW * EPT
RPT = NP // 16

_MESH = plsc.VectorSubcoreMesh(core_axis_name="c", subcore_axis_name="s")


@functools.partial(
    pl.kernel,
    out_type=jax.ShapeDtypeStruct((2, NP, D), jnp.float32),
    mesh=_MESH,
    scratch_types=[
        pltpu.VMEM((K, CHUNK), jnp.int32),
        pltpu.VMEM((CHUNK, D), jnp.float32),
        pltpu.VMEM_SHARED((NP, D), jnp.float32),
        pltpu.SemaphoreType.DMA,
        pltpu.SemaphoreType.DMA,
    ],
)
def _deg_kernel(dst_hbm, ones_hbm, zeros_hbm, out_hbm,
                didx, ones_v, acc, sem_i, sem_s):
    c = lax.axis_index("c")
    s = lax.axis_index("s")
    wid = c * 16 + s
    pltpu.sync_copy(zeros_hbm.at[pl.ds(s * RPT, RPT)], acc.at[pl.ds(s * RPT, RPT)])
    pltpu.sync_copy(ones_hbm, ones_v)
    plsc.subcore_barrier()

    ebase = wid * EPT

    def group(gi, carry):
        base = ebase + gi * (K * CHUNK)
        cps = [
            pltpu.async_copy(dst_hbm.at[pl.ds(base + b * CHUNK, CHUNK)], didx.at[b], sem_i)
            for b in range(K)
        ]
        for cp in cps:
            cp.wait()
        scs = [pltpu.async_copy(ones_v, acc.at[didx.at[b]], sem_s, add=True) for b in range(K)]
        for cp in scs:
            cp.wait()
        return carry

    lax.fori_loop(0, NSTEP // K, group, 0)
    plsc.subcore_barrier()
    pltpu.sync_copy(acc.at[pl.ds(s * RPT, RPT)], out_hbm.at[c, pl.ds(s * RPT, RPT)])


@functools.partial(
    pl.kernel,
    out_type=jax.ShapeDtypeStruct((2, NP, D), jnp.float32),
    mesh=_MESH,
    scratch_types=[
        pltpu.VMEM((K, CHUNK), jnp.int32),
        pltpu.VMEM((K, CHUNK), jnp.int32),
        pltpu.VMEM((K, CHUNK, D), jnp.float32),
        pltpu.VMEM_SHARED((NP, D), jnp.float32),
        pltpu.SemaphoreType.DMA,
        pltpu.SemaphoreType.DMA,
        pltpu.SemaphoreType.DMA,
    ],
)
def _edge_kernel(src_hbm, dst_hbm, table_hbm, zeros_hbm, out_hbm,
                 sidx, didx, rows, acc, sem_i, sem_g, sem_s):
    c = lax.axis_index("c")
    s = lax.axis_index("s")
    wid = c * 16 + s
    pltpu.sync_copy(zeros_hbm.at[pl.ds(s * RPT, RPT)], acc.at[pl.ds(s * RPT, RPT)])
    plsc.subcore_barrier()

    ebase = wid * EPT

    def group(gi, carry):
        base = ebase + gi * (K * CHUNK)
        cps = []
        for b in range(K):
            cps.append(pltpu.async_copy(src_hbm.at[pl.ds(base + b * CHUNK, CHUNK)], sidx.at[b], sem_i))
            cps.append(pltpu.async_copy(dst_hbm.at[pl.ds(base + b * CHUNK, CHUNK)], didx.at[b], sem_i))
        for cp in cps:
            cp.wait()
        gcs = [pltpu.async_copy(table_hbm.at[sidx.at[b]], rows.at[b], sem_g) for b in range(K)]
        for cp in gcs:
            cp.wait()
        scs = [pltpu.async_copy(rows.at[b], acc.at[didx.at[b]], sem_s, add=True) for b in range(K)]
        for cp in scs:
            cp.wait()
        return carry

    lax.fori_loop(0, NSTEP // K, group, 0)
    plsc.subcore_barrier()
    pltpu.sync_copy(acc.at[pl.ds(s * RPT, RPT)], out_hbm.at[c, pl.ds(s * RPT, RPT)])


def _dis(degp_ref):
    deg = degp_ref[0, :, 0:1] + degp_ref[1, :, 0:1] + 1.0
    return lax.rsqrt(deg)


def _first_body(x_ref, w_ref, degp_ref, o_ref):
    dis = _dis(degp_ref)
    g = jnp.dot(x_ref[...], w_ref[...], preferred_element_type=jnp.float32)
    o_ref[...] = g * dis


def _mid_body(s_ref, g_ref, degp_ref, w_ref, b_ref, o_ref):
    dis = _dis(degp_ref)
    h = jnp.maximum(dis * (s_ref[0] + s_ref[1] + g_ref[...]) + b_ref[...], 0.0)
    o_ref[...] = jnp.dot(h, w_ref[...], preferred_element_type=jnp.float32) * dis


def _final_body(s_ref, g_ref, degp_ref, b_ref, wc_ref, bc_ref, o_ref):
    dis = _dis(degp_ref)
    h = jnp.maximum(dis * (s_ref[0] + s_ref[1] + g_ref[...]) + b_ref[...], 0.0)
    o_ref[...] = jnp.dot(h, wc_ref[...], preferred_element_type=jnp.float32) + bc_ref[...]


_first_tc = pl.pallas_call(
    _first_body, out_shape=jax.ShapeDtypeStruct((NP, D), jnp.float32))
_mid_tc = pl.pallas_call(
    _mid_body, out_shape=jax.ShapeDtypeStruct((NP, D), jnp.float32))
_final_tc = pl.pallas_call(
    _final_body, out_shape=jax.ShapeDtypeStruct((NP, D), jnp.float32))


def kernel(x, edge_index, Ws, bs, Wc, bc):
    src = edge_index[0]
    dst = edge_index[1]
    pad = E_PAD - E
    src_p = jnp.concatenate([src, jnp.full((pad,), N, jnp.int32)])
    dst_p = jnp.concatenate([dst, jnp.full((pad,), N, jnp.int32)])

    xp = jnp.pad(x, ((0, NP - N), (0, 0)))
    zerosD = jnp.zeros((NP, D), jnp.float32)
    onesD = jnp.ones((CHUNK, D), jnp.float32)
    wc_pad = jnp.pad(Wc, ((0, 0), (0, D - Wc.shape[1])))
    bc_pad = jnp.pad(bc, (0, D - bc.shape[0])).reshape(1, D)

    degp = _deg_kernel(dst_p, onesD, zerosD)
    degs = degp[:, :, :8]

    g = _first_tc(xp, Ws[0], degs)
    for i in range(1, N_LAYERS):
        sp = _edge_kernel(src_p, dst_p, g, zerosD)
        g = _mid_tc(sp, g, degs, Ws[i], bs[i - 1].reshape(1, D))
    sp = _edge_kernel(src_p, dst_p, g, zerosD)
    out_full = _final_tc(sp, g, degs, bs[N_LAYERS - 1].reshape(1, D), wc_pad, bc_pad)
    return out_full[:N, :Wc.shape[1]]
